# baseline (speedup 1.0000x reference)
import jax
import jax.numpy as jnp
from jax import lax
from jax.experimental import pallas as pl
from jax.experimental.pallas import tpu as pltpu
from jax.experimental.pallas import tpu_sc as plsc

N_NODES = 10000
N_EDGES = 160000
D_FEAT = 256
D_EDGE = 16

LANES = 16
NW = 16
E_W = N_EDGES // NW
GROUPS = E_W // LANES
ROWS = N_EDGES // LANES


def _tc_scores_body(nf_ref, wsd_ref, efT_ref, we_ref, b_ref, s2_ref, e_ref):
    s2_ref[...] = jnp.dot(nf_ref[...], wsd_ref[...],
                          preferred_element_type=jnp.float32)
    x = efT_ref[...]
    wc = we_ref[...]
    e_ref[...] = jnp.sum(x * wc, axis=0) + b_ref[0]


def _sc_softmax_body(s2_hbm, ei_hbm, e3_hbm, out_hbm,
                     table_v, i0_v, i1_v, e_v, l_v, red_v, stage_v,
                     shared_max, shared_sum):
    wid = lax.axis_index("s")
    base = wid * E_W
    pltpu.sync_copy(s2_hbm, table_v)
    pltpu.sync_copy(ei_hbm.at[pl.ds(base, E_W)], i0_v)
    pltpu.sync_copy(ei_hbm.at[pl.ds(N_EDGES + base, E_W)], i1_v)
    pltpu.sync_copy(e3_hbm.at[pl.ds(base, E_W)], e_v)

    @plsc.parallel_loop(0, E_W, LANES, unroll=8,
                        carry=jnp.zeros((LANES,), jnp.float32))
    def m(i, acc):
        ds = pl.ds(i, LANES)
        s0 = plsc.load_gather(table_v, [i0_v[ds] * 2])
        s1 = plsc.load_gather(table_v, [i1_v[ds] * 2 + 1])
        logit = jnp.maximum(s0 + s1 + e_v[ds], 0.0)
        l_v[ds] = logit
        return jnp.maximum(acc, logit)

    stage_v[...] = m
    pltpu.sync_copy(stage_v, shared_max.at[pl.ds(wid * LANES, LANES)])
    plsc.subcore_barrier()
    pltpu.sync_copy(shared_max, red_v)
    g = red_v[pl.ds(0, LANES)]
    for i in range(1, NW):
        g = jnp.maximum(g, red_v[pl.ds(i * LANES, LANES)])
    gmax = jnp.max(g)

    @plsc.parallel_loop(0, E_W, LANES, unroll=8,
                        carry=jnp.zeros((LANES,), jnp.float32))
    def s(i, acc):
        ds = pl.ds(i, LANES)
        t = jnp.exp(l_v[ds] - gmax)
        l_v[ds] = t
        return acc + t

    stage_v[...] = s
    pltpu.sync_copy(stage_v, shared_sum.at[pl.ds(wid * LANES, LANES)])
    plsc.subcore_barrier()
    pltpu.sync_copy(shared_sum, red_v)
    a = red_v[pl.ds(0, LANES)]
    for i in range(1, NW):
        a = a + red_v[pl.ds(i * LANES, LANES)]
    inv = jnp.ones((LANES,), jnp.float32) / jnp.broadcast_to(jnp.sum(a), (LANES,))

    @plsc.parallel_loop(0, E_W, LANES, unroll=8)
    def _(i):
        ds = pl.ds(i, LANES)
        l_v[ds] = l_v[ds] * inv

    pltpu.sync_copy(l_v, out_hbm.at[pl.ds(base, E_W)])


def kernel(node_feature, edge_index, ef_init, W, b):
    ei_flat = edge_index.astype(jnp.int32).reshape(2 * N_EDGES)
    w = W[:, 0]
    wsd = jnp.stack([w[:D_FEAT], w[D_FEAT:2 * D_FEAT]], axis=1)
    we = w[2 * D_FEAT:].reshape(D_EDGE, 1)
    efT = ef_init.T

    blk_n = 5000
    blk_e = 131072
    s2, e = pl.pallas_call(
        _tc_scores_body,
        grid=(pl.cdiv(N_NODES, blk_n),),
        out_shape=(jax.ShapeDtypeStruct((N_NODES, 2), jnp.float32),
                   jax.ShapeDtypeStruct((N_EDGES,), jnp.float32)),
        in_specs=[pl.BlockSpec((blk_n, D_FEAT), lambda i: (i, 0)),
                  pl.BlockSpec((D_FEAT, 2), lambda i: (0, 0)),
                  pl.BlockSpec((D_EDGE, blk_e), lambda i: (0, i)),
                  pl.BlockSpec((D_EDGE, 1), lambda i: (0, 0)),
                  pl.BlockSpec(memory_space=pltpu.SMEM)],
        out_specs=(pl.BlockSpec((blk_n, 2), lambda i: (i, 0)),
                   pl.BlockSpec((blk_e,), lambda i: (i,))),
    )(node_feature, wsd, efT, we, b)
    table = s2.reshape(2 * N_NODES)

    mesh = plsc.VectorSubcoreMesh(core_axis_name="c", subcore_axis_name="s",
                                  num_cores=1)
    out = pl.kernel(
        _sc_softmax_body,
        out_type=jax.ShapeDtypeStruct((N_EDGES,), jnp.float32),
        mesh=mesh,
        compiler_params=pltpu.CompilerParams(needs_layout_passes=False),
        scratch_types=[
            pltpu.VMEM((2 * N_NODES,), jnp.float32),
            pltpu.VMEM((E_W,), jnp.int32),
            pltpu.VMEM((E_W,), jnp.int32),
            pltpu.VMEM((E_W,), jnp.float32),
            pltpu.VMEM((E_W,), jnp.float32),
            pltpu.VMEM((NW * LANES,), jnp.float32),
            pltpu.VMEM((LANES,), jnp.float32),
            pltpu.VMEM_SHARED((NW * LANES,), jnp.float32),
            pltpu.VMEM_SHARED((NW * LANES,), jnp.float32),
        ],
    )(table, ei_flat, e)
    return out


# speedup vs baseline: 1.0102x; 1.0102x over previous
import jax
import jax.numpy as jnp
from jax import lax
from jax.experimental import pallas as pl
from jax.experimental.pallas import tpu as pltpu
from jax.experimental.pallas import tpu_sc as plsc

N_NODES = 10000
N_EDGES = 160000
D_FEAT = 256
D_EDGE = 16

LANES = 16
NW = 16
E_W = N_EDGES // NW
GROUPS = E_W // LANES
ROWS = N_EDGES // LANES


def _tc_scores_body(nf_ref, wsd_ref, efT_ref, we_ref, b_ref, s2_ref, e_ref):
    s2_ref[...] = jnp.dot(nf_ref[...], wsd_ref[...],
                          preferred_element_type=jnp.float32)
    x = efT_ref[...]
    wc = we_ref[...]
    e_ref[...] = jnp.sum(x * wc, axis=0) + b_ref[0]


def _sc_softmax_body(s2_hbm, ei_hbm, e3_hbm, out_hbm,
                     table_v, i0_v, i1_v, e_v, l_v, red_v, stage_v,
                     shared_max, shared_sum):
    wid = lax.axis_index("s")
    base = wid * E_W
    pltpu.sync_copy(s2_hbm, table_v)
    pltpu.sync_copy(ei_hbm.at[pl.ds(base, E_W)], i0_v)
    pltpu.sync_copy(ei_hbm.at[pl.ds(N_EDGES + base, E_W)], i1_v)
    pltpu.sync_copy(e3_hbm.at[pl.ds(base, E_W)], e_v)

    @plsc.parallel_loop(0, E_W, LANES, unroll=8,
                        carry=jnp.zeros((LANES,), jnp.float32))
    def m(i, acc):
        ds = pl.ds(i, LANES)
        s0 = plsc.load_gather(table_v, [i0_v[ds] * 2])
        s1 = plsc.load_gather(table_v, [i1_v[ds] * 2 + 1])
        logit = jnp.maximum(s0 + s1 + e_v[ds], 0.0)
        l_v[ds] = logit
        return jnp.maximum(acc, logit)

    stage_v[...] = m
    pltpu.sync_copy(stage_v, shared_max.at[pl.ds(wid * LANES, LANES)])
    plsc.subcore_barrier()
    pltpu.sync_copy(shared_max, red_v)
    g = red_v[pl.ds(0, LANES)]
    for i in range(1, NW):
        g = jnp.maximum(g, red_v[pl.ds(i * LANES, LANES)])
    gmax = jnp.max(g)

    @plsc.parallel_loop(0, E_W, LANES, unroll=8,
                        carry=jnp.zeros((LANES,), jnp.float32))
    def s(i, acc):
        ds = pl.ds(i, LANES)
        t = jnp.exp(l_v[ds] - gmax)
        l_v[ds] = t
        return acc + t

    stage_v[...] = s
    pltpu.sync_copy(stage_v, shared_sum.at[pl.ds(wid * LANES, LANES)])
    plsc.subcore_barrier()
    pltpu.sync_copy(shared_sum, red_v)
    a = red_v[pl.ds(0, LANES)]
    for i in range(1, NW):
        a = a + red_v[pl.ds(i * LANES, LANES)]
    inv = jnp.ones((LANES,), jnp.float32) / jnp.broadcast_to(jnp.sum(a), (LANES,))

    @plsc.parallel_loop(0, E_W, LANES, unroll=8)
    def _(i):
        ds = pl.ds(i, LANES)
        l_v[ds] = l_v[ds] * inv

    pltpu.sync_copy(l_v, out_hbm.at[pl.ds(base, E_W)])


def kernel(node_feature, edge_index, ef_init, W, b):
    ei_flat = edge_index.astype(jnp.int32).reshape(2 * N_EDGES)
    w = W[:, 0]
    wsd = jnp.stack([w[:D_FEAT], w[D_FEAT:2 * D_FEAT]], axis=1)
    we = w[2 * D_FEAT:].reshape(D_EDGE, 1)
    efT = ef_init.T

    blk_n = 2000
    blk_e = 32768
    s2, e = pl.pallas_call(
        _tc_scores_body,
        grid=(pl.cdiv(N_NODES, blk_n),),
        out_shape=(jax.ShapeDtypeStruct((N_NODES, 2), jnp.float32),
                   jax.ShapeDtypeStruct((N_EDGES,), jnp.float32)),
        in_specs=[pl.BlockSpec((blk_n, D_FEAT), lambda i: (i, 0)),
                  pl.BlockSpec((D_FEAT, 2), lambda i: (0, 0)),
                  pl.BlockSpec((D_EDGE, blk_e), lambda i: (0, i)),
                  pl.BlockSpec((D_EDGE, 1), lambda i: (0, 0)),
                  pl.BlockSpec(memory_space=pltpu.SMEM)],
        out_specs=(pl.BlockSpec((blk_n, 2), lambda i: (i, 0)),
                   pl.BlockSpec((blk_e,), lambda i: (i,))),
    )(node_feature, wsd, efT, we, b)
    table = s2.reshape(2 * N_NODES)

    mesh = plsc.VectorSubcoreMesh(core_axis_name="c", subcore_axis_name="s",
                                  num_cores=1)
    out = pl.kernel(
        _sc_softmax_body,
        out_type=jax.ShapeDtypeStruct((N_EDGES,), jnp.float32),
        mesh=mesh,
        compiler_params=pltpu.CompilerParams(needs_layout_passes=False),
        scratch_types=[
            pltpu.VMEM((2 * N_NODES,), jnp.float32),
            pltpu.VMEM((E_W,), jnp.int32),
            pltpu.VMEM((E_W,), jnp.int32),
            pltpu.VMEM((E_W,), jnp.float32),
            pltpu.VMEM((E_W,), jnp.float32),
            pltpu.VMEM((NW * LANES,), jnp.float32),
            pltpu.VMEM((LANES,), jnp.float32),
            pltpu.VMEM_SHARED((NW * LANES,), jnp.float32),
            pltpu.VMEM_SHARED((NW * LANES,), jnp.float32),
        ],
    )(table, ei_flat, e)
    return out
